# Initial kernel scaffold; baseline (speedup 1.0000x reference)
import jax
import jax.numpy as jnp
from jax.experimental import pallas as pl


def kernel(x, gating_output, gate_up_proj, down_proj):
    raise NotImplementedError("write your pallas kernel here")


# speedup vs baseline: 1.1200x; 1.1200x over previous
import functools

import jax
import jax.numpy as jnp
from jax import lax
from jax.experimental import pallas as pl
from jax.experimental.pallas import tpu as pltpu
from jax.experimental.pallas import tpu_sc as plsc

E = 8
K = 2
D = 1024
F = 2048
T = 2048
B = 256
NB = (T * K) // B + E
S = NB * B

NC = 2
NS = 16
NW = NC * NS


def _router_body(g_ref, e1_ref, e2_ref, w1_ref, w2_ref):
    s = g_ref[...]
    ii = lax.broadcasted_iota(jnp.int32, (T, E), 1)
    m1 = jnp.max(s, axis=1, keepdims=True)
    i1 = jnp.min(jnp.where(s == m1, ii, E), axis=1, keepdims=True)
    s2 = jnp.where(ii == i1, -jnp.inf, s)
    m2 = jnp.max(s2, axis=1, keepdims=True)
    i2 = jnp.min(jnp.where(s2 == m2, ii, E), axis=1, keepdims=True)
    e1_ref[...] = i1
    e2_ref[...] = i2
    w1_ref[...] = jax.nn.sigmoid(m1 - m2)
    w2_ref[...] = jax.nn.sigmoid(m2 - m1)


def _router(gating):
    return pl.pallas_call(
        _router_body,
        out_shape=(
            jax.ShapeDtypeStruct((T, 1), jnp.int32),
            jax.ShapeDtypeStruct((T, 1), jnp.int32),
            jax.ShapeDtypeStruct((T, 1), jnp.float32),
            jax.ShapeDtypeStruct((T, 1), jnp.float32),
        ),
    )(gating)


def _routing_metadata(e1, e2, w1, w2):
    ea = jnp.concatenate([e1, e2], axis=1).reshape(T * K)
    wa = jnp.concatenate([w1, w2], axis=1).reshape(T * K)
    order = jnp.argsort(ea)
    e_sorted = ea[order]
    g = jnp.bincount(ea, length=E)
    c = (g + B - 1) // B
    starts = jnp.concatenate([jnp.zeros(1, jnp.int32), jnp.cumsum(g)[:-1].astype(jnp.int32)])
    pstarts = jnp.concatenate([jnp.zeros(1, jnp.int32), jnp.cumsum(c * B)[:-1].astype(jnp.int32)])
    ii = jnp.arange(T * K, dtype=jnp.int32)
    slot_sorted = pstarts[e_sorted] + (ii - starts[e_sorted])
    row_of_slot = jnp.zeros((S,), jnp.int32).at[slot_sorted].set((order // K).astype(jnp.int32))
    w_of_slot = jnp.zeros((S,), jnp.float32).at[slot_sorted].set(wa[order])
    inv_slot = jnp.zeros((T * K,), jnp.int32).at[order].set(slot_sorted)
    p0 = inv_slot[0::2]
    p1 = inv_slot[1::2]
    block_expert = jnp.repeat(
        jnp.arange(E, dtype=jnp.int32), c, total_repeat_length=NB)
    nb_real = jnp.sum(c).astype(jnp.int32)
    valid = (jnp.arange(NB, dtype=jnp.int32) < nb_real).astype(jnp.int32)
    return row_of_slot, w_of_slot.reshape(NB, B, 1), p0, p1, block_expert, valid


_PER_W = S // NW
_CH = 96


def _dispatch(x, row_of_slot):
    mesh = plsc.VectorSubcoreMesh(
        core_axis_name="c", subcore_axis_name="s", num_cores=NC, num_subcores=NS)

    @functools.partial(
        pl.kernel,
        out_type=jax.ShapeDtypeStruct((S, D), jnp.float32),
        mesh=mesh,
        scratch_types=[
            pltpu.VMEM((_CH,), jnp.int32),
            pltpu.VMEM((_CH, D), jnp.float32),
            pltpu.SemaphoreType.DMA,
        ],
    )
    def k(x_hbm, rows_hbm, out_hbm, idx_v, buf, sem):
        wid = lax.axis_index("s") * NC + lax.axis_index("c")
        base = wid * _PER_W
        for ci in range(_PER_W // _CH):
            off = base + ci * _CH
            pltpu.sync_copy(rows_hbm.at[pl.ds(off, _CH)], idx_v)
            pltpu.async_copy(x_hbm.at[idx_v], buf, sem).wait()
            pltpu.sync_copy(buf, out_hbm.at[pl.ds(off, _CH)])

    return k(x, row_of_slot)


def _ffn_body(be_ref, va_ref, xs_ref, gup_ref, down_ref, w_ref, ys_ref):
    b = pl.program_id(0)

    @pl.when(va_ref[b] != 0)
    def _():
        xb = xs_ref[...]
        gup = gup_ref[0]
        acc = lax.dot_general(xb, gup, (((1,), (1,)), ((), ())),
                              preferred_element_type=jnp.float32)
        gte = acc[:, :F]
        up = acc[:, F:]
        h = gte * jax.nn.sigmoid(gte) * up
        dwn = down_ref[0]
        y = lax.dot_general(h, dwn, (((1,), (1,)), ((), ())),
                            preferred_element_type=jnp.float32)
        ys_ref[...] = y * w_ref[0]


def _ffn(xs, gup, down, w_blocks, block_expert, valid):
    grid_spec = pltpu.PrefetchScalarGridSpec(
        num_scalar_prefetch=2,
        grid=(NB,),
        in_specs=[
            pl.BlockSpec((B, D), lambda b, be, va: (b, 0)),
            pl.BlockSpec((1, 2 * F, D), lambda b, be, va: (be[b], 0, 0)),
            pl.BlockSpec((1, D, F), lambda b, be, va: (be[b], 0, 0)),
            pl.BlockSpec((1, B, 1), lambda b, be, va: (b, 0, 0)),
        ],
        out_specs=pl.BlockSpec((B, D), lambda b, be, va: (b, 0)),
    )
    return pl.pallas_call(
        _ffn_body,
        grid_spec=grid_spec,
        out_shape=jax.ShapeDtypeStruct((S, D), jnp.float32),
    )(block_expert, valid, xs, gup, down, w_blocks)


_PER_T = T // NW
_CHT = 32


def _combine(ys, p0, p1):
    mesh = plsc.VectorSubcoreMesh(
        core_axis_name="c", subcore_axis_name="s", num_cores=NC, num_subcores=NS)

    @functools.partial(
        pl.kernel,
        out_type=jax.ShapeDtypeStruct((T, D), jnp.float32),
        mesh=mesh,
        scratch_types=[
            pltpu.VMEM((_CHT,), jnp.int32),
            pltpu.VMEM((_CHT,), jnp.int32),
            pltpu.VMEM((_CHT, D), jnp.float32),
            pltpu.VMEM((_CHT, D), jnp.float32),
            pltpu.SemaphoreType.DMA,
        ],
    )
    def k(ys_hbm, p0_hbm, p1_hbm, out_hbm, i0_v, i1_v, buf0, buf1, sem):
        wid = lax.axis_index("s") * NC + lax.axis_index("c")
        base = wid * _PER_T
        for ci in range(_PER_T // _CHT):
            off = base + ci * _CHT
            pltpu.sync_copy(p0_hbm.at[pl.ds(off, _CHT)], i0_v)
            pltpu.sync_copy(p1_hbm.at[pl.ds(off, _CHT)], i1_v)
            cp0 = pltpu.async_copy(ys_hbm.at[i0_v], buf0, sem)
            cp1 = pltpu.async_copy(ys_hbm.at[i1_v], buf1, sem)
            cp0.wait()
            cp1.wait()

            def row_add(r, _):
                for j in range(D // 16):
                    sl = (r, pl.ds(j * 16, 16))
                    buf0[sl] = buf0[sl] + buf1[sl]
                return 0

            lax.fori_loop(0, _CHT, row_add, 0)
            pltpu.sync_copy(buf0, out_hbm.at[pl.ds(off, _CHT)])

    return k(ys, p0, p1)


def kernel(x, gating_output, gate_up_proj, down_proj):
    e1, e2, w1, w2 = _router(gating_output)
    row_of_slot, w_blocks, p0, p1, block_expert, valid = _routing_metadata(
        e1, e2, w1, w2)
    xs = _dispatch(x, row_of_slot)
    ys = _ffn(xs, gate_up_proj, down_proj, w_blocks, block_expert, valid)
    return _combine(ys, p0, p1)
